# baseline (device time: 10954 ns/iter reference)
import jax
import jax.numpy as jnp
from jax import lax
from jax.experimental import pallas as pl
from jax.experimental.pallas import tpu as pltpu

T = 256
D = 512
V_SHARD = 4096
N_CHUNKS = 8
VC = V_SHARD // N_CHUNKS


def kernel(x, W, labels):

    def body(x_hbm, w_hbm, labels_hbm, out_hbm,
             x_vmem, w_vmem, labels_vmem, stats_ref, stats2_ref,
             rstats_ref, rstats2_ref, out_vmem,
             in_sems, out_sem, send_sem, recv_sem, send_sem2, recv_sem2):
        my_x = lax.axis_index("x")
        my_y = lax.axis_index("y")
        my_z = lax.axis_index("z")
        partner = (1 - my_x, my_y, my_z)

        barrier_sem = pltpu.get_barrier_semaphore()
        pl.semaphore_signal(
            barrier_sem, inc=1,
            device_id=partner, device_id_type=pl.DeviceIdType.MESH,
        )

        cp_x = pltpu.make_async_copy(x_hbm, x_vmem, in_sems.at[N_CHUNKS])
        cp_lab = pltpu.make_async_copy(
            labels_hbm, labels_vmem, in_sems.at[N_CHUNKS + 1]
        )
        cp_x.start()
        cp_lab.start()
        cp_w = []
        for c in range(N_CHUNKS):
            sl = pl.ds(c * VC, VC)
            cp = pltpu.make_async_copy(
                w_hbm.at[:, sl], w_vmem.at[:, sl], in_sems.at[c]
            )
            cp.start()
            cp_w.append(cp)

        cp_x.wait()
        cp_lab.wait()
        xv = x_vmem[:, :]
        local_idx = labels_vmem[:] - my_x * V_SHARD

        HALF = N_CHUNKS // 2
        col = lax.broadcasted_iota(jnp.int32, (T, VC), 1)

        def stats_over(chunks):
            s = jnp.zeros((T,), jnp.float32)
            ll = jnp.zeros((T,), jnp.float32)
            for c in chunks:
                cp_w[c].wait()
                logits = jnp.dot(
                    xv, w_vmem[:, c * VC:(c + 1) * VC],
                    preferred_element_type=jnp.float32,
                )
                s = s + jnp.sum(jnp.exp(logits), axis=1)
                ll = ll + jnp.sum(
                    jnp.where(
                        col == (local_idx - c * VC)[:, None], logits, 0.0
                    ),
                    axis=1,
                )
            return s, ll

        s1, ll1 = stats_over(range(HALF))
        stats_ref[0, :] = s1
        stats_ref[1, :] = ll1
        pl.semaphore_wait(barrier_sem, 1)
        rdma1 = pltpu.make_async_remote_copy(
            src_ref=stats_ref,
            dst_ref=rstats_ref,
            send_sem=send_sem,
            recv_sem=recv_sem,
            device_id=partner,
            device_id_type=pl.DeviceIdType.MESH,
        )
        rdma1.start()

        s2, ll2 = stats_over(range(HALF, N_CHUNKS))
        stats2_ref[0, :] = s2
        stats2_ref[1, :] = ll2
        rdma2 = pltpu.make_async_remote_copy(
            src_ref=stats2_ref,
            dst_ref=rstats2_ref,
            send_sem=send_sem2,
            recv_sem=recv_sem2,
            device_id=partner,
            device_id_type=pl.DeviceIdType.MESH,
        )
        rdma2.start()

        rdma1.wait_recv()
        rdma2.wait_recv()
        out_vmem[:] = (
            jnp.log(s1 + s2 + rstats_ref[0, :] + rstats2_ref[0, :])
            - (ll1 + ll2 + rstats_ref[1, :] + rstats2_ref[1, :])
        )
        cp_out = pltpu.make_async_copy(out_vmem, out_hbm, out_sem)
        cp_out.start()
        rdma1.wait_send()
        rdma2.wait_send()
        cp_out.wait()

    x = pltpu.with_memory_space_constraint(x, pltpu.HBM)
    W = pltpu.with_memory_space_constraint(W, pltpu.HBM)
    labels = pltpu.with_memory_space_constraint(labels, pltpu.HBM)

    return pl.pallas_call(
        body,
        out_shape=jax.ShapeDtypeStruct((T,), jnp.float32),
        in_specs=[
            pl.BlockSpec(memory_space=pltpu.HBM),
            pl.BlockSpec(memory_space=pltpu.HBM),
            pl.BlockSpec(memory_space=pltpu.HBM),
        ],
        out_specs=pl.BlockSpec(memory_space=pltpu.HBM),
        scratch_shapes=[
            pltpu.VMEM((T, D), jnp.float32),
            pltpu.VMEM((D, V_SHARD), jnp.float32),
            pltpu.VMEM((T,), jnp.int32),
            pltpu.VMEM((2, T), jnp.float32),
            pltpu.VMEM((2, T), jnp.float32),
            pltpu.VMEM((2, T), jnp.float32),
            pltpu.VMEM((2, T), jnp.float32),
            pltpu.VMEM((T,), jnp.float32),
            pltpu.SemaphoreType.DMA((N_CHUNKS + 2,)),
            pltpu.SemaphoreType.DMA,
            pltpu.SemaphoreType.DMA,
            pltpu.SemaphoreType.DMA,
            pltpu.SemaphoreType.DMA,
            pltpu.SemaphoreType.DMA,
        ],
        compiler_params=pltpu.CompilerParams(collective_id=0),
    )(x, W, labels)


# device time: 9735 ns/iter; 1.1252x vs baseline; 1.1252x over previous
import jax
import jax.numpy as jnp
from jax import lax
from jax.experimental import pallas as pl
from jax.experimental.pallas import tpu as pltpu

T = 256
D = 512
V_SHARD = 4096
N_CHUNKS = 4
VC = V_SHARD // N_CHUNKS


def kernel(x, W, labels):

    def body(x_hbm, w_hbm, labels_hbm, out_hbm,
             x_vmem, w_vmem, labels_vmem, stats_ref, rstats_ref, out_vmem,
             in_sems, out_sem, send_sem, recv_sem):
        my_x = lax.axis_index("x")
        my_y = lax.axis_index("y")
        my_z = lax.axis_index("z")
        partner = (1 - my_x, my_y, my_z)

        barrier_sem = pltpu.get_barrier_semaphore()
        pl.semaphore_signal(
            barrier_sem, inc=1,
            device_id=partner, device_id_type=pl.DeviceIdType.MESH,
        )

        cp_x = pltpu.make_async_copy(x_hbm, x_vmem, in_sems.at[N_CHUNKS])
        cp_lab = pltpu.make_async_copy(
            labels_hbm, labels_vmem, in_sems.at[N_CHUNKS + 1]
        )
        cp_x.start()
        cp_lab.start()
        cp_w = []
        for c in range(N_CHUNKS):
            sl = pl.ds(c * VC, VC)
            cp = pltpu.make_async_copy(
                w_hbm.at[:, sl], w_vmem.at[:, sl], in_sems.at[c]
            )
            cp.start()
            cp_w.append(cp)

        cp_x.wait()
        cp_lab.wait()
        xv = x_vmem[:, :]
        local_idx = labels_vmem[:] - my_x * V_SHARD

        col = lax.broadcasted_iota(jnp.int32, (T, VC), 1)
        s = jnp.zeros((T,), jnp.float32)
        ll = jnp.zeros((T,), jnp.float32)
        for c in range(N_CHUNKS):
            cp_w[c].wait()
            logits = jnp.dot(
                xv, w_vmem[:, c * VC:(c + 1) * VC],
                preferred_element_type=jnp.float32,
            )
            s = s + jnp.sum(jnp.exp(logits), axis=1)
            ll = ll + jnp.sum(
                jnp.where(col == (local_idx - c * VC)[:, None], logits, 0.0),
                axis=1,
            )

        stats_ref[0, :] = s
        stats_ref[1, :] = ll

        pl.semaphore_wait(barrier_sem, 1)
        rdma = pltpu.make_async_remote_copy(
            src_ref=stats_ref,
            dst_ref=rstats_ref,
            send_sem=send_sem,
            recv_sem=recv_sem,
            device_id=partner,
            device_id_type=pl.DeviceIdType.MESH,
        )
        rdma.start()
        rdma.wait_recv()

        out_vmem[:] = (
            jnp.log(s + rstats_ref[0, :]) - (ll + rstats_ref[1, :])
        )
        cp_out = pltpu.make_async_copy(out_vmem, out_hbm, out_sem)
        cp_out.start()
        rdma.wait_send()
        cp_out.wait()

    x = pltpu.with_memory_space_constraint(x, pltpu.HBM)
    W = pltpu.with_memory_space_constraint(W, pltpu.HBM)
    labels = pltpu.with_memory_space_constraint(labels, pltpu.HBM)

    return pl.pallas_call(
        body,
        out_shape=jax.ShapeDtypeStruct((T,), jnp.float32),
        in_specs=[
            pl.BlockSpec(memory_space=pltpu.HBM),
            pl.BlockSpec(memory_space=pltpu.HBM),
            pl.BlockSpec(memory_space=pltpu.HBM),
        ],
        out_specs=pl.BlockSpec(memory_space=pltpu.HBM),
        scratch_shapes=[
            pltpu.VMEM((T, D), jnp.float32),
            pltpu.VMEM((D, V_SHARD), jnp.float32),
            pltpu.VMEM((T,), jnp.int32),
            pltpu.VMEM((2, T), jnp.float32),
            pltpu.VMEM((2, T), jnp.float32),
            pltpu.VMEM((T,), jnp.float32),
            pltpu.SemaphoreType.DMA((N_CHUNKS + 2,)),
            pltpu.SemaphoreType.DMA,
            pltpu.SemaphoreType.DMA,
            pltpu.SemaphoreType.DMA,
        ],
        compiler_params=pltpu.CompilerParams(collective_id=0),
    )(x, W, labels)
